# Initial kernel scaffold; baseline (speedup 1.0000x reference)
import jax
import jax.numpy as jnp
from jax.experimental import pallas as pl


def kernel(x, W, b, W_r, b_r):
    raise NotImplementedError("write your pallas kernel here")


# speedup vs baseline: 16.9653x; 16.9653x over previous
import functools

import jax
import jax.numpy as jnp
from jax.experimental import pallas as pl
from jax.experimental.pallas import tpu as pltpu

_IN = 2048
_OUT = 2048
_TOPK = max(1, int(_OUT * 0.1))
_BT = 256


def _float_keys(r):
    bits = jax.lax.bitcast_convert_type(r, jnp.int32)
    return bits ^ ((bits >> 31) & jnp.int32(0x7FFFFFFF))


def _kth_largest_keys(keys, k):
    rows = keys.shape[0]
    lo = jnp.full((rows, 1), jnp.iinfo(jnp.int32).min, jnp.int32)
    hi = jnp.full((rows, 1), jnp.iinfo(jnp.int32).max, jnp.int32)

    def body(_, carry):
        lo, hi = carry
        mid = (lo >> 1) + (hi >> 1) + ((lo | hi) & 1)
        cnt = jnp.sum((keys >= mid).astype(jnp.int32), axis=1, keepdims=True)
        ge = cnt >= k
        return jnp.where(ge, mid, lo), jnp.where(ge, hi, mid - 1)

    lo, _ = jax.lax.fori_loop(0, 32, body, (lo, hi))
    return lo


def _fused_kernel(x_ref, wr_ref, br_ref, w_ref, b_ref, out_ref):
    xb = x_ref[...]
    dims = (((1,), (1,)), ((), ()))
    r = jax.lax.dot_general(xb, wr_ref[...], dims,
                            preferred_element_type=jnp.float32) + br_ref[...]
    keys = _float_keys(r)
    kth = _kth_largest_keys(keys, _TOPK)
    mask = (keys >= kth).astype(jnp.float32)
    o = jax.lax.dot_general(xb, w_ref[...], dims,
                            preferred_element_type=jnp.float32) + b_ref[...]
    out_ref[...] = o * mask


@jax.jit
def kernel(x, W, b, W_r, b_r):
    B, S, F = x.shape
    T = B * S
    xt = x.reshape(T, F)
    grid = (T // _BT,)
    out = pl.pallas_call(
        _fused_kernel,
        grid=grid,
        in_specs=[
            pl.BlockSpec((_BT, F), lambda i: (i, 0)),
            pl.BlockSpec((_OUT, F), lambda i: (0, 0)),
            pl.BlockSpec((1, _OUT), lambda i: (0, 0)),
            pl.BlockSpec((_OUT, F), lambda i: (0, 0)),
            pl.BlockSpec((1, _OUT), lambda i: (0, 0)),
        ],
        out_specs=pl.BlockSpec((_BT, _OUT), lambda i: (i, 0)),
        out_shape=jax.ShapeDtypeStruct((T, _OUT), jnp.float32),
    )(xt, W_r, b_r.reshape(1, _OUT), W, b.reshape(1, _OUT))
    return out.reshape(B, S, _OUT)
